# baseline (speedup 1.0000x reference)
import functools

import jax
import jax.numpy as jnp
from jax import lax
from jax.experimental import pallas as pl
from jax.experimental.pallas import tpu as pltpu, tpu_sc as plsc

_NC, _NS, _LANES = 2, 16, 16
_NW = _NC * _NS

_B, _L, _D = 4, 2048, 1024
_LPW = _L // _NW
_CROWS = 32
_CW = _CROWS * _D


def _sc_body(x_hbm, pe_hbm, o_hbm, xbuf, pebuf):
    wid = lax.axis_index("s") * _NC + lax.axis_index("c")
    lbase = wid * _LPW
    for h in range(_LPW // _CROWS):
        l0 = lbase + h * _CROWS
        pltpu.sync_copy(pe_hbm.at[pl.ds(pl.multiple_of((l0 + 1) * _D, _D), _CW)],
                        pebuf)
        for b in range(_B):
            off = pl.multiple_of((b * _L + l0) * _D, _D)
            pltpu.sync_copy(x_hbm.at[pl.ds(off, _CW)], xbuf)

            @plsc.parallel_loop(0, _CW, step=_LANES, unroll=8)
            def _add(i):
                xbuf[pl.ds(i, _LANES)] = (
                    xbuf[pl.ds(i, _LANES)] + pebuf[pl.ds(i, _LANES)])

            pltpu.sync_copy(xbuf, o_hbm.at[pl.ds(off, _CW)])


def _sc_kernel(x, pe):
    b, l, d = x.shape
    mesh = plsc.VectorSubcoreMesh(
        core_axis_name="c", subcore_axis_name="s",
        num_cores=_NC, num_subcores=_NS)
    out = pl.kernel(
        _sc_body,
        out_type=jax.ShapeDtypeStruct((b * l * d,), x.dtype),
        mesh=mesh,
        scratch_types=[
            pltpu.VMEM((_CW,), jnp.float32),
            pltpu.VMEM((_CW,), jnp.float32),
        ],
    )(x.reshape(-1), pe.reshape(-1))
    return out.reshape(b, l, d)


def _tc_body(x_ref, pe_a, pe_b, o_ref):
    rows = jnp.concatenate([pe_a[1:, :], pe_b[:1, :]], axis=0)
    o_ref[...] = x_ref[...] + rows[None, :, :]


def _tc_kernel(x, pe):
    b, l, d = x.shape
    cl = 256
    return pl.pallas_call(
        _tc_body,
        grid=(l // cl,),
        in_specs=[
            pl.BlockSpec((b, cl, d), lambda i: (0, i, 0)),
            pl.BlockSpec((cl, d), lambda i: (i, 0)),
            pl.BlockSpec((cl, d), lambda i: (i + 1, 0)),
        ],
        out_specs=pl.BlockSpec((b, cl, d), lambda i: (0, i, 0)),
        out_shape=jax.ShapeDtypeStruct((b, l, d), x.dtype),
    )(x, pe, pe)


def kernel(x, pe):
    return _sc_kernel(x, pe)


# speedup vs baseline: 1.1645x; 1.1645x over previous
import functools

import jax
import jax.numpy as jnp
from jax import lax
from jax.experimental import pallas as pl
from jax.experimental.pallas import tpu as pltpu, tpu_sc as plsc

_NC, _NS, _LANES = 2, 16, 16
_NW = _NC * _NS

_B, _L, _D = 4, 2048, 1024
_LPW = _L // _NW
_CROWS = 16
_CW = _CROWS * _D
_NCH = _B * (_LPW // _CROWS)
_NXB = 4
_NPB = 2


def _sc_body(x_hbm, pe_hbm, o_hbm, xbufs, pbufs, sxs, sps, sos):
    wid = lax.axis_index("s") * _NC + lax.axis_index("c")
    lbase = wid * _LPW
    nh = _LPW // _CROWS

    def x_slice(k):
        b, h = k % _B, k // _B
        off = pl.multiple_of(b * _L * _D + (lbase + h * _CROWS) * _D, _D)
        return pl.ds(off, _CW)

    def start_x(k):
        return pltpu.async_copy(x_hbm.at[x_slice(k)], xbufs[k % _NXB],
                                sxs[k % _NXB])

    def start_pe(h):
        off = pl.multiple_of((lbase + h * _CROWS + 1) * _D, _D)
        return pltpu.async_copy(pe_hbm.at[pl.ds(off, _CW)], pbufs[h % _NPB],
                                sps[h % _NPB])

    pe_d = {h: start_pe(h) for h in range(min(_NPB, nh))}
    x_d = {k: start_x(k) for k in range(min(_NXB - 2, _NCH))}
    out_d = {}
    for k in range(_NCH):
        nk = k + _NXB - 2
        if nk < _NCH:
            if nk - _NXB >= 0:
                out_d[nk - _NXB].wait()
            x_d[nk] = start_x(nk)
        h = k // _B
        if k % _B == 0:
            pe_d[h].wait()
        x_d[k].wait()
        xbuf, pbuf = xbufs[k % _NXB], pbufs[h % _NPB]

        @plsc.parallel_loop(0, _CW, step=_LANES, unroll=8)
        def _add(i):
            xbuf[pl.ds(i, _LANES)] = (
                xbuf[pl.ds(i, _LANES)] + pbuf[pl.ds(i, _LANES)])

        out_d[k] = pltpu.async_copy(xbuf, o_hbm.at[x_slice(k)], sos[k % _NXB])
        if k % _B == _B - 1 and h + _NPB < nh:
            pe_d[h + _NPB] = start_pe(h + _NPB)
    for k in range(max(0, _NCH - _NXB), _NCH):
        out_d[k].wait()


def _sc_kernel(x, pe):
    b, l, d = x.shape
    mesh = plsc.VectorSubcoreMesh(
        core_axis_name="c", subcore_axis_name="s",
        num_cores=_NC, num_subcores=_NS)
    out = pl.kernel(
        _sc_body,
        out_type=jax.ShapeDtypeStruct((b * l * d,), x.dtype),
        mesh=mesh,
        scratch_types=[
            [pltpu.VMEM((_CW,), jnp.float32) for _ in range(_NXB)],
            [pltpu.VMEM((_CW,), jnp.float32) for _ in range(_NPB)],
            [pltpu.SemaphoreType.DMA for _ in range(_NXB)],
            [pltpu.SemaphoreType.DMA for _ in range(_NPB)],
            [pltpu.SemaphoreType.DMA for _ in range(_NXB)],
        ],
    )(x.reshape(-1), pe.reshape(-1))
    return out.reshape(b, l, d)


def _tc_body(x_ref, pe_a, pe_b, o_ref):
    rows = jnp.concatenate([pe_a[1:, :], pe_b[:1, :]], axis=0)
    o_ref[...] = x_ref[...] + rows[None, :, :]


def _tc_kernel(x, pe):
    b, l, d = x.shape
    cl = 256
    return pl.pallas_call(
        _tc_body,
        grid=(l // cl,),
        in_specs=[
            pl.BlockSpec((b, cl, d), lambda i: (0, i, 0)),
            pl.BlockSpec((cl, d), lambda i: (i, 0)),
            pl.BlockSpec((cl, d), lambda i: (i + 1, 0)),
        ],
        out_specs=pl.BlockSpec((b, cl, d), lambda i: (0, i, 0)),
        out_shape=jax.ShapeDtypeStruct((b, l, d), x.dtype),
    )(x, pe, pe)


def kernel(x, pe):
    return _sc_kernel(x, pe)


# speedup vs baseline: 2.3997x; 2.0608x over previous
import functools

import jax
import jax.numpy as jnp
from jax import lax
from jax.experimental import pallas as pl
from jax.experimental.pallas import tpu as pltpu, tpu_sc as plsc

_NC, _NS, _LANES = 2, 16, 16
_NW = _NC * _NS

_B, _L, _D = 4, 2048, 1024
_LPW = _L // _NW
_CROWS = 16
_CW = _CROWS * _D
_NCH = _B * (_LPW // _CROWS)
_NXB = 4
_NPB = 2


def _sc_body(x_hbm, pe_hbm, o_hbm, xbufs, pbufs, sxs, sps, sos):
    wid = lax.axis_index("s") * _NC + lax.axis_index("c")
    lbase = wid * _LPW
    nh = _LPW // _CROWS

    def x_slice(k):
        b, h = k % _B, k // _B
        row = pl.multiple_of(b * _L + lbase + h * _CROWS, 8)
        return pl.ds(row, _CROWS)

    def start_x(k):
        return pltpu.async_copy(x_hbm.at[x_slice(k)], xbufs[k % _NXB],
                                sxs[k % _NXB])

    def pe_slice(h):
        row = pl.multiple_of(lbase + h * _CROWS, 8)
        return pl.ds(row, _CROWS + 8)

    def start_pe(h, slot):
        pltpu.async_copy(pe_hbm.at[pe_slice(h)], pbufs[slot], sps[slot])

    start_pe(0, 0)
    start_pe(1, 1)
    start_x(0)
    start_x(1)

    def half(g, carry):
        for j in range(8):
            k = g * 8 + j
            if j % 4 == 0:
                slot = j // 4
                pltpu.make_async_copy(pe_hbm.at[pe_slice(2 * g + slot)],
                                      pbufs[slot], sps[slot]).wait()
            pltpu.make_async_copy(x_hbm.at[x_slice(k)], xbufs[j % 4],
                                  sxs[j % 4]).wait()
            xbuf, pbuf = xbufs[j % 4], pbufs[j // 4]
            for r in range(_CROWS):

                @plsc.parallel_loop(0, _D, step=_LANES, unroll=8)
                def _add(c):
                    xbuf[r, pl.ds(c, _LANES)] = (
                        xbuf[r, pl.ds(c, _LANES)]
                        + pbuf[r + 1, pl.ds(c, _LANES)])

            pltpu.async_copy(xbuf, o_hbm.at[x_slice(k)], sos[j % 4])
            nslot = (j + 2) % 4

            def wait_prev_out():
                pltpu.make_async_copy(xbufs[nslot], o_hbm.at[x_slice(k)],
                                      sos[nslot]).wait()

            if j < 2:
                pl.when(g > 0)(wait_prev_out)
                pltpu.async_copy(x_hbm.at[x_slice(k + 2)], xbufs[nslot],
                                 sxs[nslot])
            elif j < 6:
                wait_prev_out()
                pltpu.async_copy(x_hbm.at[x_slice(k + 2)], xbufs[nslot],
                                 sxs[nslot])
            else:
                @pl.when(g < 1)
                def _():
                    wait_prev_out()
                    pltpu.async_copy(x_hbm.at[x_slice(k + 2)], xbufs[nslot],
                                     sxs[nslot])
            if j % 4 == 3:
                slot = j // 4

                @pl.when(g < 1)
                def _():
                    start_pe(2 * g + 2 + slot, slot)
        return carry

    lax.fori_loop(0, 2, half, 0)
    for j in range(4):
        pltpu.make_async_copy(xbufs[j], o_hbm.at[x_slice(12 + j)],
                              sos[j]).wait()


def _sc_kernel(x, pe):
    b, l, d = x.shape
    mesh = plsc.VectorSubcoreMesh(
        core_axis_name="c", subcore_axis_name="s",
        num_cores=_NC, num_subcores=_NS)
    out = pl.kernel(
        _sc_body,
        out_type=jax.ShapeDtypeStruct((b * l, d), x.dtype),
        mesh=mesh,
        scratch_types=[
            [pltpu.VMEM((_CROWS, _D), jnp.float32) for _ in range(_NXB)],
            [pltpu.VMEM((_CROWS + 8, _D), jnp.float32) for _ in range(_NPB)],
            [pltpu.SemaphoreType.DMA for _ in range(_NXB)],
            [pltpu.SemaphoreType.DMA for _ in range(_NPB)],
            [pltpu.SemaphoreType.DMA for _ in range(_NXB)],
        ],
    )(x.reshape(b * l, d), pe)
    return out.reshape(b, l, d)


def _tc_body(x_ref, pe_a, pe_b, o_ref):
    rows = jnp.concatenate([pe_a[1:, :], pe_b[:1, :]], axis=0)
    o_ref[...] = x_ref[...] + rows[None, :, :]


def _tc_kernel(x, pe):
    b, l, d = x.shape
    cl = 256
    return pl.pallas_call(
        _tc_body,
        grid=(l // cl,),
        in_specs=[
            pl.BlockSpec((b, cl, d), lambda i: (0, i, 0)),
            pl.BlockSpec((cl, d), lambda i: (i, 0)),
            pl.BlockSpec((cl, d), lambda i: (i + 1, 0)),
        ],
        out_specs=pl.BlockSpec((b, cl, d), lambda i: (0, i, 0)),
        out_shape=jax.ShapeDtypeStruct((b, l, d), x.dtype),
    )(x, pe, pe)


def kernel(x, pe):
    return _sc_kernel(x, pe)


# speedup vs baseline: 2.4032x; 1.0014x over previous
import jax
import jax.numpy as jnp
from jax import lax
from jax.experimental import pallas as pl
from jax.experimental.pallas import tpu as pltpu, tpu_sc as plsc

_NC, _NS, _LANES = 2, 16, 16
_NW = _NC * _NS

_B, _L, _D = 4, 2048, 1024
_LPW = _L // _NW
_CROWS = 16
_CW = _CROWS * _D
_NCH = _B * (_LPW // _CROWS)
_NXB = 4
_NPB = 2


def _sc_body(x_hbm, pe_hbm, o_hbm, xbufs, pbufs, sxs, sps, sos):
    wid = lax.axis_index("s") * _NC + lax.axis_index("c")
    lbase = wid * _LPW
    nh = _LPW // _CROWS

    def x_slice(k):
        b, h = k % _B, k // _B
        row = pl.multiple_of(b * _L + lbase + h * _CROWS, 8)
        return pl.ds(row, _CROWS)

    def start_x(k):
        return pltpu.async_copy(x_hbm.at[x_slice(k)], xbufs[k % _NXB],
                                sxs[k % _NXB])

    def pe_slice(h):
        row = pl.multiple_of(lbase + h * _CROWS, 8)
        return pl.ds(row, _CROWS + 8)

    def start_pe(h, slot):
        pltpu.async_copy(pe_hbm.at[pe_slice(h)], pbufs[slot], sps[slot])

    start_pe(0, 0)
    start_pe(1, 1)
    start_x(0)
    start_x(1)

    def half(g, carry):
        for j in range(8):
            k = g * 8 + j
            if j % 4 == 0:
                slot = j // 4
                pltpu.make_async_copy(pe_hbm.at[pe_slice(2 * g + slot)],
                                      pbufs[slot], sps[slot]).wait()
            pltpu.make_async_copy(x_hbm.at[x_slice(k)], xbufs[j % 4],
                                  sxs[j % 4]).wait()
            xbuf, pbuf = xbufs[j % 4], pbufs[j // 4]
            for r in range(_CROWS):

                @plsc.parallel_loop(0, _D, step=_LANES, unroll=8)
                def _add(c):
                    xbuf[r, pl.ds(c, _LANES)] = (
                        xbuf[r, pl.ds(c, _LANES)]
                        + pbuf[r + 1, pl.ds(c, _LANES)])

            pltpu.async_copy(xbuf, o_hbm.at[x_slice(k)], sos[j % 4])
            nslot = (j + 2) % 4

            def wait_prev_out():
                pltpu.make_async_copy(xbufs[nslot], o_hbm.at[x_slice(k)],
                                      sos[nslot]).wait()

            if j < 2:
                pl.when(g > 0)(wait_prev_out)
                pltpu.async_copy(x_hbm.at[x_slice(k + 2)], xbufs[nslot],
                                 sxs[nslot])
            elif j < 6:
                wait_prev_out()
                pltpu.async_copy(x_hbm.at[x_slice(k + 2)], xbufs[nslot],
                                 sxs[nslot])
            else:
                @pl.when(g < 1)
                def _():
                    wait_prev_out()
                    pltpu.async_copy(x_hbm.at[x_slice(k + 2)], xbufs[nslot],
                                     sxs[nslot])
            if j % 4 == 3:
                slot = j // 4

                @pl.when(g < 1)
                def _():
                    start_pe(2 * g + 2 + slot, slot)
        return carry

    lax.fori_loop(0, 2, half, 0)
    for j in range(4):
        pltpu.make_async_copy(xbufs[j], o_hbm.at[x_slice(12 + j)],
                              sos[j]).wait()


def _sc_kernel(x, pe):
    b, l, d = x.shape
    mesh = plsc.VectorSubcoreMesh(
        core_axis_name="c", subcore_axis_name="s",
        num_cores=_NC, num_subcores=_NS)
    out = pl.kernel(
        _sc_body,
        out_type=jax.ShapeDtypeStruct((b * l, d), x.dtype),
        mesh=mesh,
        scratch_types=[
            [pltpu.VMEM((_CROWS, _D), jnp.float32) for _ in range(_NXB)],
            [pltpu.VMEM((_CROWS + 8, _D), jnp.float32) for _ in range(_NPB)],
            [pltpu.SemaphoreType.DMA for _ in range(_NXB)],
            [pltpu.SemaphoreType.DMA for _ in range(_NPB)],
            [pltpu.SemaphoreType.DMA for _ in range(_NXB)],
        ],
    )(x.reshape(b * l, d), pe)
    return out.reshape(b, l, d)


def kernel(x, pe):
    return _sc_kernel(x, pe)
